# baseline (speedup 1.0000x reference)
import jax
import jax.numpy as jnp
from jax.experimental import pallas as pl

N = 10000; E = 160000; F_IN = 256; HID = 512; NL = 3; NEXP = 4; NCLS = 10; NGRAPH = 64


def _gumbel(key, shape):
    u = jax.random.uniform(key, shape, dtype=jnp.float32)
    return -jnp.log(-jnp.log(u + 1e-20) + 1e-20)


def kernel(x, edge_index, batch, ce_W1_0, ce_b1_0, ce_W2_0, ce_b2_0, ce_W1, ce_b1, ce_W2, ce_b2, ce_eps, cl_W1_0, cl_b1_0, cl_W2_0, cl_b2_0, cl_W1, cl_b1, cl_W2, cl_b2, cl_eps, Wn1, bn1, Wn2, bn2, We1, be1, We2, be2, Wf1, bf1, Wf2, bf2, Wc, bc):
    src = edge_index[0]; dst = edge_index[1]

    def gin(h, W1_0, b1_0, W2_0, b2_0, W1, b1, W2, b2, eps, edge_weight=None):
        for l in range(NL):
            msg = h[src]
            if edge_weight is not None:
                msg = msg * edge_weight[:, None]
            agg = jax.ops.segment_sum(msg, dst, num_segments=N)
            h2 = (1.0 + eps[l]) * h + agg
            if l == 0:
                h2 = jnp.maximum(h2 @ W1_0 + b1_0, 0.0) @ W2_0 + b2_0
            else:
                h2 = jnp.maximum(h2 @ W1[l - 1] + b1[l - 1], 0.0) @ W2[l - 1] + b2[l - 1]
            h = jnp.maximum(h2, 0.0) if l < NL - 1 else h2
        return h

    def gmp(h):
        s = jax.ops.segment_sum(h, batch, num_segments=NGRAPH)
        cnt = jax.ops.segment_sum(jnp.ones((h.shape[0], 1), jnp.float32), batch, num_segments=NGRAPH)
        return s / jnp.maximum(cnt, 1.0)

    Z = gin(x, ce_W1_0, ce_b1_0, ce_W2_0, ce_b2_0, ce_W1, ce_b1, ce_W2, ce_b2, ce_eps)

    mkey = jax.random.key(42)
    g_n = [_gumbel(jax.random.fold_in(mkey, 3 * i), (N, 1)) for i in range(NEXP)]
    g_e = [_gumbel(jax.random.fold_in(mkey, 3 * i + 1), (E, 1)) for i in range(NEXP)]
    g_f = [_gumbel(jax.random.fold_in(mkey, 3 * i + 2), (N, F_IN)) for i in range(NEXP)]

    logits_list = []; hst_list = []; nm_list = []; em_list = []; fm_list = []
    for i in range(NEXP):
        nlog = jnp.maximum(Z @ Wn1[i] + bn1[i], 0.0) @ Wn2[i] + bn2[i]
        A = Z @ We1[i][:HID]; B = Z @ We1[i][HID:]
        ehid = jnp.maximum(A[src] + B[dst] + be1[i], 0.0)
        elog = ehid @ We2[i] + be2[i]
        flog = jnp.maximum(Z @ Wf1[i] + bf1[i], 0.0) @ Wf2[i] + bf2[i]
        nmask = (nlog + g_n[i] > 0.0).astype(jnp.float32)
        emask = (elog + g_e[i] > 0.0).astype(jnp.float32)
        fmask = (flog + g_f[i] > 0.0).astype(jnp.float32)
        masked_x = x * nmask * fmask
        ew = emask.reshape(-1)
        mZ = gin(masked_x, cl_W1_0, cl_b1_0, cl_W2_0, cl_b2_0, cl_W1, cl_b1, cl_W2, cl_b2, cl_eps, edge_weight=ew)
        hst = gmp(mZ)
        logits_list.append(hst @ Wc[i] + bc[i])
        hst_list.append(hst); nm_list.append(nmask); em_list.append(emask); fm_list.append(fmask)

    expert_logits = jnp.stack(logits_list, axis=1)
    h_stable = jnp.stack(hst_list, axis=1)
    node_masks = jnp.stack(nm_list, axis=1)
    edge_masks = jnp.stack(em_list, axis=1)
    feat_masks = jnp.stack(fm_list, axis=1)
    h_orig = gmp(Z)
    return expert_logits, h_stable, h_orig, node_masks, edge_masks, feat_masks


# speedup vs baseline: 1.4497x; 1.4497x over previous
import functools

import jax
import jax.numpy as jnp
from jax import lax
from jax.experimental import pallas as pl
from jax.experimental.pallas import tpu as pltpu
from jax.experimental.pallas import tpu_sc as plsc

N = 10000; E = 160000; F_IN = 256; HID = 512; NL = 3; NEXP = 4; NCLS = 10; NGRAPH = 64

NPAD = 10240
DUMMY0 = N
CH = 128
EPT = 10112
EPAD = 16 * EPT


def _seg_pass(h_flat, src_pad, dst_pad, C, NLIST):
    C2 = C // 2
    CPL = C // NLIST

    mesh = plsc.VectorSubcoreMesh(core_axis_name="c", subcore_axis_name="s")

    @functools.partial(
        pl.kernel, mesh=mesh,
        out_type=jax.ShapeDtypeStruct((C * N, 128), jnp.float32),
        scratch_types=[
            pltpu.VMEM((CH,), jnp.int32),
            pltpu.VMEM((CH,), jnp.int32),
            pltpu.VMEM((CH, 128), jnp.float32),
            pltpu.VMEM((16, 128), jnp.float32),
            pltpu.VMEM_SHARED((NPAD, 128), jnp.float32),
            pltpu.SemaphoreType.DMA,
        ],
    )
    def k(h_hbm, src_hbm, dst_hbm, out_hbm, sidx, didx, rows, zbuf, acc, sem):
        c = lax.axis_index("c")
        s = lax.axis_index("s")
        zero16 = jnp.zeros((16,), jnp.float32)
        for i in range(16):
            for j in range(8):
                zbuf[i, pl.ds(j * 16, 16)] = zero16
        ebase = s * EPT

        for kk in range(C2):
            chunk = c * C2 + kk
            list_id = chunk // CPL

            def zbody(j, _):
                pltpu.sync_copy(zbuf, acc.at[pl.ds(s * 640 + j * 16, 16)])
                return 0
            lax.fori_loop(0, 40, zbody, 0)
            plsc.subcore_barrier()

            def ebody(g, _):
                off = ebase + g * CH
                pltpu.sync_copy(src_hbm.at[pl.ds(off, CH)], sidx)

                def adj(j, _):
                    sidx[pl.ds(j * 16, 16)] = sidx[pl.ds(j * 16, 16)] + chunk * N
                    return 0
                lax.fori_loop(0, CH // 16, adj, 0)
                pltpu.sync_copy(dst_hbm.at[pl.ds(list_id * EPAD + off, CH)], didx)
                pltpu.async_copy(h_hbm.at[sidx], rows, sem).wait()
                pltpu.sync_copy(rows, acc.at[didx], add=True)
                return 0
            lax.fori_loop(0, EPT // CH, ebody, 0)
            plsc.subcore_barrier()

            offr = jnp.minimum(s * 632, N - 632)
            pltpu.sync_copy(acc.at[pl.ds(offr, 632)],
                            out_hbm.at[pl.ds(chunk * N + offr, 632)])
            plsc.subcore_barrier()

    return k(h_flat, src_pad, dst_pad)


def _to_chunk_major(h):
    n, f = h.shape
    c = f // 128
    return h.reshape(n, c, 128).transpose(1, 0, 2).reshape(c * n, 128)


def _from_chunk_major(hf, c):
    return hf.reshape(c, N, 128).transpose(1, 0, 2).reshape(N, c * 128)


def _gumbel(key, shape):
    u = jax.random.uniform(key, shape, dtype=jnp.float32)
    return -jnp.log(-jnp.log(u + 1e-20) + 1e-20)


def kernel(x, edge_index, batch, ce_W1_0, ce_b1_0, ce_W2_0, ce_b2_0, ce_W1, ce_b1, ce_W2, ce_b2, ce_eps, cl_W1_0, cl_b1_0, cl_W2_0, cl_b2_0, cl_W1, cl_b1, cl_W2, cl_b2, cl_eps, Wn1, bn1, Wn2, bn2, We1, be1, We2, be2, Wf1, bf1, Wf2, bf2, Wc, bc):
    src = edge_index[0]; dst = edge_index[1]
    order = jnp.argsort(dst, stable=True)
    src_s = src[order]; dst_s = dst[order]
    npad = EPAD - E
    src_pad = jnp.concatenate([src_s, jnp.zeros((npad,), jnp.int32)])
    dummy = DUMMY0 + (jnp.arange(E, dtype=jnp.int32) % 16)
    dst_pad_plain = jnp.concatenate(
        [dst_s, jnp.full((npad,), DUMMY0, jnp.int32)])

    def gin(h, W1_0, b1_0, W2_0, b2_0, W1, b1, W2, b2, eps, dst_lists=None, nexp=1):
        for l in range(NL):
            f = h.shape[1] // nexp
            cpe = f // 128
            hf = _to_chunk_major(h)
            dl = dst_pad_plain if dst_lists is None else dst_lists
            if dst_lists is None:
                nlist = 1
            else:
                nlist = nexp
            aggf = _seg_pass(hf, src_pad, dl, cpe * nexp, nlist)
            agg = _from_chunk_major(aggf, cpe * nexp)
            h2 = (1.0 + eps[l]) * h + agg
            if nexp > 1:
                h2 = h2.reshape(N, nexp, f)
                if l == 0:
                    h2 = jnp.einsum("nef,fh->neh", jnp.maximum(jnp.einsum("nef,fh->neh", h2, W1_0) + b1_0, 0.0), W2_0) + b2_0
                else:
                    h2 = jnp.einsum("nef,fh->neh", jnp.maximum(jnp.einsum("nef,fh->neh", h2, W1[l - 1]) + b1[l - 1], 0.0), W2[l - 1]) + b2[l - 1]
                h2 = h2.reshape(N, nexp * HID)
            else:
                if l == 0:
                    h2 = jnp.maximum(h2 @ W1_0 + b1_0, 0.0) @ W2_0 + b2_0
                else:
                    h2 = jnp.maximum(h2 @ W1[l - 1] + b1[l - 1], 0.0) @ W2[l - 1] + b2[l - 1]
            h = jnp.maximum(h2, 0.0) if l < NL - 1 else h2
        return h

    def gmp(h):
        s = jax.ops.segment_sum(h, batch, num_segments=NGRAPH)
        cnt = jax.ops.segment_sum(jnp.ones((h.shape[0], 1), jnp.float32), batch, num_segments=NGRAPH)
        return s / jnp.maximum(cnt, 1.0)

    Z = gin(x, ce_W1_0, ce_b1_0, ce_W2_0, ce_b2_0, ce_W1, ce_b1, ce_W2, ce_b2, ce_eps)

    mkey = jax.random.key(42)
    g_n = [_gumbel(jax.random.fold_in(mkey, 3 * i), (N, 1)) for i in range(NEXP)]
    g_e = [_gumbel(jax.random.fold_in(mkey, 3 * i + 1), (E, 1)) for i in range(NEXP)]
    g_f = [_gumbel(jax.random.fold_in(mkey, 3 * i + 2), (N, F_IN)) for i in range(NEXP)]

    logits_list = []; hst_list = []; nm_list = []; em_list = []; fm_list = []
    masked_xs = []; dst_lists = []
    for i in range(NEXP):
        nlog = jnp.maximum(Z @ Wn1[i] + bn1[i], 0.0) @ Wn2[i] + bn2[i]
        A = Z @ We1[i][:HID]; B = Z @ We1[i][HID:]
        ehid = jnp.maximum(A[src] + B[dst] + be1[i], 0.0)
        elog = ehid @ We2[i] + be2[i]
        flog = jnp.maximum(Z @ Wf1[i] + bf1[i], 0.0) @ Wf2[i] + bf2[i]
        nmask = (nlog + g_n[i] > 0.0).astype(jnp.float32)
        emask = (elog + g_e[i] > 0.0).astype(jnp.float32)
        fmask = (flog + g_f[i] > 0.0).astype(jnp.float32)
        masked_xs.append(x * nmask * fmask)
        emask_s = emask.reshape(-1)[order]
        dst_adj = jnp.where(emask_s > 0.0, dst_s, dummy)
        dst_lists.append(jnp.concatenate([dst_adj, jnp.full((npad,), DUMMY0, jnp.int32)]))
        nm_list.append(nmask); em_list.append(emask); fm_list.append(fmask)

    h_all = jnp.concatenate(masked_xs, axis=1)
    dl_all = jnp.concatenate(dst_lists)
    mZ_all = gin(h_all, cl_W1_0, cl_b1_0, cl_W2_0, cl_b2_0, cl_W1, cl_b1, cl_W2, cl_b2, cl_eps,
                 dst_lists=dl_all, nexp=NEXP)
    for i in range(NEXP):
        hst = gmp(mZ_all[:, i * HID:(i + 1) * HID])
        logits_list.append(hst @ Wc[i] + bc[i])
        hst_list.append(hst)

    expert_logits = jnp.stack(logits_list, axis=1)
    h_stable = jnp.stack(hst_list, axis=1)
    node_masks = jnp.stack(nm_list, axis=1)
    edge_masks = jnp.stack(em_list, axis=1)
    feat_masks = jnp.stack(fm_list, axis=1)
    h_orig = gmp(Z)
    return expert_logits, h_stable, h_orig, node_masks, edge_masks, feat_masks


# speedup vs baseline: 1.5042x; 1.0376x over previous
import functools

import jax
import jax.numpy as jnp
from jax import lax
from jax.experimental import pallas as pl
from jax.experimental.pallas import tpu as pltpu
from jax.experimental.pallas import tpu_sc as plsc

N = 10000; E = 160000; F_IN = 256; HID = 512; NL = 3; NEXP = 4; NCLS = 10; NGRAPH = 64

NPAD = 10240
DUMMY0 = N
CH = 128
EPT = 10240
EPAD = 16 * EPT


def _seg_pass(h_flat, src_pad, dst_pad, C, NLIST):
    C2 = C // 2
    CPL = C // NLIST
    NIT = EPT // CH
    NH = NIT // 2

    mesh = plsc.VectorSubcoreMesh(core_axis_name="c", subcore_axis_name="s")

    @functools.partial(
        pl.kernel, mesh=mesh,
        out_type=jax.ShapeDtypeStruct((C * N, 128), jnp.float32),
        scratch_types=[
            pltpu.VMEM((NH, CH), jnp.int32),
            pltpu.VMEM((NH, CH), jnp.int32),
            pltpu.VMEM((CH, 128), jnp.float32),
            pltpu.VMEM((CH, 128), jnp.float32),
            pltpu.VMEM((32, 128), jnp.float32),
            pltpu.VMEM_SHARED((NPAD, 128), jnp.float32),
            pltpu.SemaphoreType.DMA,
            pltpu.SemaphoreType.DMA,
            pltpu.SemaphoreType.DMA,
        ],
    )
    def k(h_hbm, src_hbm, dst_hbm, out_hbm, sidx, didx, rows0, rows1,
          zbuf, acc, sem0, sem1, zsem):
        c = lax.axis_index("c")
        s = lax.axis_index("s")
        zero16 = jnp.zeros((16,), jnp.float32)

        def zfill(j, _):
            zbuf[j // 8, pl.ds((j % 8) * 16, 16)] = zero16
            return 0
        lax.fori_loop(0, 256, zfill, 0)
        rows = (rows0, rows1)
        sems = (sem0, sem1)

        for kk in range(C2):
            chunk = c * C2 + kk
            list_id = chunk // CPL

            zh = [pltpu.async_copy(zbuf, acc.at[pl.ds(s * 640 + j * 32, 32)], zsem)
                  for j in range(20)]
            for h in zh:
                h.wait()
            plsc.subcore_barrier()

            for hh in range(2):
                pltpu.sync_copy(src_hbm.at[pl.ds(s * NIT + hh * NH, NH)], sidx)
                pltpu.sync_copy(
                    dst_hbm.at[pl.ds(list_id * (EPAD // CH) + s * NIT + hh * NH, NH)],
                    didx)

                def adj(j, _):
                    r = j // (CH // 16)
                    col = (j % (CH // 16)) * 16
                    sidx[r, pl.ds(col, 16)] = sidx[r, pl.ds(col, 16)] + chunk * N
                    return 0
                lax.fori_loop(0, NH * (CH // 16), adj, 0)

                dmas = [None, None]
                dmas[0] = pltpu.async_copy(h_hbm.at[sidx.at[0]], rows[0], sems[0])
                for g in range(NH):
                    sl = g % 2
                    if g + 1 < NH:
                        nsl = (g + 1) % 2
                        dmas[nsl] = pltpu.async_copy(
                            h_hbm.at[sidx.at[g + 1]], rows[nsl], sems[nsl])
                    dmas[sl].wait()
                    pltpu.sync_copy(rows[sl], acc.at[didx.at[g]], add=True)
            plsc.subcore_barrier()

            offr = jnp.minimum(s * 632, N - 632)
            pltpu.sync_copy(acc.at[pl.ds(offr, 632)],
                            out_hbm.at[pl.ds(chunk * N + offr, 632)])
            plsc.subcore_barrier()

    return k(h_flat, src_pad.reshape(EPAD // CH, CH), dst_pad.reshape(NLIST * (EPAD // CH), CH))


def _to_chunk_major(h):
    n, f = h.shape
    c = f // 128
    return h.reshape(n, c, 128).transpose(1, 0, 2).reshape(c * n, 128)


def _from_chunk_major(hf, c):
    return hf.reshape(c, N, 128).transpose(1, 0, 2).reshape(N, c * 128)


def _gumbel(key, shape):
    u = jax.random.uniform(key, shape, dtype=jnp.float32)
    return -jnp.log(-jnp.log(u + 1e-20) + 1e-20)


def kernel(x, edge_index, batch, ce_W1_0, ce_b1_0, ce_W2_0, ce_b2_0, ce_W1, ce_b1, ce_W2, ce_b2, ce_eps, cl_W1_0, cl_b1_0, cl_W2_0, cl_b2_0, cl_W1, cl_b1, cl_W2, cl_b2, cl_eps, Wn1, bn1, Wn2, bn2, We1, be1, We2, be2, Wf1, bf1, Wf2, bf2, Wc, bc):
    src = edge_index[0]; dst = edge_index[1]
    order = jnp.argsort(dst, stable=True)
    src_s = src[order]; dst_s = dst[order]
    npad = EPAD - E
    src_pad = jnp.concatenate([src_s, jnp.zeros((npad,), jnp.int32)])
    dummy = DUMMY0 + (jnp.arange(E, dtype=jnp.int32) % 16)
    dst_pad_plain = jnp.concatenate(
        [dst_s, jnp.full((npad,), DUMMY0, jnp.int32)])

    def gin(h, W1_0, b1_0, W2_0, b2_0, W1, b1, W2, b2, eps, dst_lists=None, nexp=1):
        for l in range(NL):
            f = h.shape[1] // nexp
            cpe = f // 128
            hf = _to_chunk_major(h)
            dl = dst_pad_plain if dst_lists is None else dst_lists
            if dst_lists is None:
                nlist = 1
            else:
                nlist = nexp
            aggf = _seg_pass(hf, src_pad, dl, cpe * nexp, nlist)
            agg = _from_chunk_major(aggf, cpe * nexp)
            h2 = (1.0 + eps[l]) * h + agg
            if nexp > 1:
                h2 = h2.reshape(N, nexp, f)
                if l == 0:
                    h2 = jnp.einsum("nef,fh->neh", jnp.maximum(jnp.einsum("nef,fh->neh", h2, W1_0) + b1_0, 0.0), W2_0) + b2_0
                else:
                    h2 = jnp.einsum("nef,fh->neh", jnp.maximum(jnp.einsum("nef,fh->neh", h2, W1[l - 1]) + b1[l - 1], 0.0), W2[l - 1]) + b2[l - 1]
                h2 = h2.reshape(N, nexp * HID)
            else:
                if l == 0:
                    h2 = jnp.maximum(h2 @ W1_0 + b1_0, 0.0) @ W2_0 + b2_0
                else:
                    h2 = jnp.maximum(h2 @ W1[l - 1] + b1[l - 1], 0.0) @ W2[l - 1] + b2[l - 1]
            h = jnp.maximum(h2, 0.0) if l < NL - 1 else h2
        return h

    def gmp(h):
        s = jax.ops.segment_sum(h, batch, num_segments=NGRAPH)
        cnt = jax.ops.segment_sum(jnp.ones((h.shape[0], 1), jnp.float32), batch, num_segments=NGRAPH)
        return s / jnp.maximum(cnt, 1.0)

    Z = gin(x, ce_W1_0, ce_b1_0, ce_W2_0, ce_b2_0, ce_W1, ce_b1, ce_W2, ce_b2, ce_eps)

    mkey = jax.random.key(42)
    g_n = [_gumbel(jax.random.fold_in(mkey, 3 * i), (N, 1)) for i in range(NEXP)]
    g_e = [_gumbel(jax.random.fold_in(mkey, 3 * i + 1), (E, 1)) for i in range(NEXP)]
    g_f = [_gumbel(jax.random.fold_in(mkey, 3 * i + 2), (N, F_IN)) for i in range(NEXP)]

    logits_list = []; hst_list = []; nm_list = []; em_list = []; fm_list = []
    masked_xs = []; dst_lists = []
    for i in range(NEXP):
        nlog = jnp.maximum(Z @ Wn1[i] + bn1[i], 0.0) @ Wn2[i] + bn2[i]
        A = Z @ We1[i][:HID]; B = Z @ We1[i][HID:]
        ehid = jnp.maximum(A[src] + B[dst] + be1[i], 0.0)
        elog = ehid @ We2[i] + be2[i]
        flog = jnp.maximum(Z @ Wf1[i] + bf1[i], 0.0) @ Wf2[i] + bf2[i]
        nmask = (nlog + g_n[i] > 0.0).astype(jnp.float32)
        emask = (elog + g_e[i] > 0.0).astype(jnp.float32)
        fmask = (flog + g_f[i] > 0.0).astype(jnp.float32)
        masked_xs.append(x * nmask * fmask)
        emask_s = emask.reshape(-1)[order]
        dst_adj = jnp.where(emask_s > 0.0, dst_s, dummy)
        dst_lists.append(jnp.concatenate([dst_adj, jnp.full((npad,), DUMMY0, jnp.int32)]))
        nm_list.append(nmask); em_list.append(emask); fm_list.append(fmask)

    h_all = jnp.concatenate(masked_xs, axis=1)
    dl_all = jnp.concatenate(dst_lists)
    mZ_all = gin(h_all, cl_W1_0, cl_b1_0, cl_W2_0, cl_b2_0, cl_W1, cl_b1, cl_W2, cl_b2, cl_eps,
                 dst_lists=dl_all, nexp=NEXP)
    for i in range(NEXP):
        hst = gmp(mZ_all[:, i * HID:(i + 1) * HID])
        logits_list.append(hst @ Wc[i] + bc[i])
        hst_list.append(hst)

    expert_logits = jnp.stack(logits_list, axis=1)
    h_stable = jnp.stack(hst_list, axis=1)
    node_masks = jnp.stack(nm_list, axis=1)
    edge_masks = jnp.stack(em_list, axis=1)
    feat_masks = jnp.stack(fm_list, axis=1)
    h_orig = gmp(Z)
    return expert_logits, h_stable, h_orig, node_masks, edge_masks, feat_masks


# speedup vs baseline: 1.5134x; 1.0062x over previous
import functools

import jax
import jax.numpy as jnp
from jax import lax
from jax.experimental import pallas as pl
from jax.experimental.pallas import tpu as pltpu
from jax.experimental.pallas import tpu_sc as plsc

N = 10000; E = 160000; F_IN = 256; HID = 512; NL = 3; NEXP = 4; NCLS = 10; NGRAPH = 64

NPAD = 10240
DUMMY0 = N
CH = 128
EPT = 10240
EPAD = 16 * EPT


def _seg_pass(h_flat, src_pad, dst_pad, C, NLIST):
    C2 = C // 2
    CPL = C // NLIST
    NIT = EPT // CH
    NH = NIT // 2

    mesh = plsc.VectorSubcoreMesh(core_axis_name="c", subcore_axis_name="s")

    @functools.partial(
        pl.kernel, mesh=mesh,
        out_type=jax.ShapeDtypeStruct((C * N, 128), jnp.float32),
        scratch_types=[
            pltpu.VMEM((NH, CH), jnp.int32),
            pltpu.VMEM((NH, CH), jnp.int32),
            pltpu.VMEM((CH, 128), jnp.float32),
            pltpu.VMEM((CH, 128), jnp.float32),
            pltpu.VMEM((32, 128), jnp.float32),
            pltpu.VMEM_SHARED((NPAD, 128), jnp.float32),
            pltpu.SemaphoreType.DMA,
            pltpu.SemaphoreType.DMA,
            pltpu.SemaphoreType.DMA,
        ],
    )
    def k(h_hbm, src_hbm, dst_hbm, out_hbm, sidx, didx, rows0, rows1,
          zbuf, acc, sem0, sem1, zsem):
        c = lax.axis_index("c")
        s = lax.axis_index("s")
        zero16 = jnp.zeros((16,), jnp.float32)

        def zfill(j, _):
            zbuf[j // 8, pl.ds((j % 8) * 16, 16)] = zero16
            return 0
        lax.fori_loop(0, 256, zfill, 0)
        rows = (rows0, rows1)
        sems = (sem0, sem1)

        for kk in range(C2):
            chunk = c * C2 + kk
            list_id = chunk // CPL

            zh = [pltpu.async_copy(zbuf, acc.at[pl.ds(s * 640 + j * 32, 32)], zsem)
                  for j in range(20)]
            for h in zh:
                h.wait()
            plsc.subcore_barrier()

            for hh in range(2):
                pltpu.sync_copy(src_hbm.at[pl.ds(s * NIT + hh * NH, NH)], sidx)
                pltpu.sync_copy(
                    dst_hbm.at[pl.ds(list_id * (EPAD // CH) + s * NIT + hh * NH, NH)],
                    didx)

                def adj(j, _):
                    r = j // (CH // 16)
                    col = (j % (CH // 16)) * 16
                    sidx[r, pl.ds(col, 16)] = sidx[r, pl.ds(col, 16)] + chunk * N
                    return 0
                lax.fori_loop(0, NH * (CH // 16), adj, 0)

                for g in range(NH):
                    sl = g % 2
                    pltpu.async_copy(h_hbm.at[sidx.at[g]], rows[sl], sems[sl]).wait()
                    pltpu.sync_copy(rows[sl], acc.at[didx.at[g]], add=True)
            plsc.subcore_barrier()

            offr = jnp.minimum(s * 632, N - 632)
            pltpu.sync_copy(acc.at[pl.ds(offr, 632)],
                            out_hbm.at[pl.ds(chunk * N + offr, 632)])
            plsc.subcore_barrier()

    return k(h_flat, src_pad.reshape(EPAD // CH, CH), dst_pad.reshape(NLIST * (EPAD // CH), CH))


def _to_chunk_major(h):
    n, f = h.shape
    c = f // 128
    return h.reshape(n, c, 128).transpose(1, 0, 2).reshape(c * n, 128)


def _from_chunk_major(hf, c):
    return hf.reshape(c, N, 128).transpose(1, 0, 2).reshape(N, c * 128)


RB = 400
NB = N // RB


def _gin_mlp(h_cm, agg_cm, eps, W1, b1, W2, b2, relu_out, nexp):
    cin = W1.shape[0] // 128
    cout = W2.shape[1] // 128
    eps = eps.reshape(1, 1)

    def body(eps_ref, h_ref, agg_ref, W1_ref, b1_ref, W2_ref, b2_ref, o_ref):
        scale = 1.0 + eps_ref[0, 0]
        acc = jnp.zeros((RB, W1_ref.shape[1]), jnp.float32)
        for c in range(cin):
            h2c = scale * h_ref[c] + agg_ref[c]
            acc = acc + jnp.dot(h2c, W1_ref[pl.ds(c * 128, 128), :],
                                preferred_element_type=jnp.float32)
        hid = jnp.maximum(acc + b1_ref[0], 0.0)
        out = jnp.dot(hid, W2_ref[...], preferred_element_type=jnp.float32) + b2_ref[0]
        if relu_out:
            out = jnp.maximum(out, 0.0)
        for c in range(cout):
            o_ref[c] = out[:, c * 128:(c + 1) * 128]

    return pl.pallas_call(
        body,
        grid=(nexp, NB),
        in_specs=[
            pl.BlockSpec(memory_space=pltpu.SMEM),
            pl.BlockSpec((cin, RB, 128), lambda i, r: (i, r, 0)),
            pl.BlockSpec((cin, RB, 128), lambda i, r: (i, r, 0)),
            pl.BlockSpec((cin * 128, W1.shape[1]), lambda i, r: (0, 0)),
            pl.BlockSpec((1, W1.shape[1]), lambda i, r: (0, 0)),
            pl.BlockSpec((W2.shape[0], cout * 128), lambda i, r: (0, 0)),
            pl.BlockSpec((1, cout * 128), lambda i, r: (0, 0)),
        ],
        out_specs=pl.BlockSpec((cout, RB, 128), lambda i, r: (i, r, 0)),
        out_shape=jax.ShapeDtypeStruct((nexp * cout, N, 128), jnp.float32),
    )(eps, h_cm, agg_cm, W1, b1.reshape(1, -1), W2, b2.reshape(1, -1))


def _expert_heads(Z_cm, x, Wn1, bn1, Wn2, bn2, We1, Wf1, bf1, Wf2, bf2,
                  gn, gf):
    CZ = HID // 128
    CX = F_IN // 128

    def body(Z_ref, x_ref, Wn1_ref, bn1_ref, Wn2_ref, bn2_ref, We1_ref,
             Wf1_ref, bf1_ref, Wf2_ref, bf2_ref, gn_ref, gf_ref,
             nm_ref, fm_ref, mx_ref, P_ref, Q_ref):
        def zmat(W_ref, r0):
            acc = jnp.zeros((RB, W_ref.shape[2]), jnp.float32)
            for c in range(CZ):
                acc = acc + jnp.dot(Z_ref[c], W_ref[0, pl.ds(r0 + c * 128, 128), :],
                                    preferred_element_type=jnp.float32)
            return acc

        hidN = jnp.maximum(zmat(Wn1_ref, 0) + bn1_ref[0], 0.0)
        nlog = jnp.dot(hidN, Wn2_ref[0], preferred_element_type=jnp.float32) + bn2_ref[0]
        nm = (nlog + gn_ref[0] > 0.0).astype(jnp.float32)
        hidF = jnp.maximum(zmat(Wf1_ref, 0) + bf1_ref[0], 0.0)
        flog = jnp.dot(hidF, Wf2_ref[0], preferred_element_type=jnp.float32) + bf2_ref[0]
        fm = (flog + gf_ref[0] > 0.0).astype(jnp.float32)
        mx = x_ref[...] * nm * fm
        nm_ref[0] = nm
        fm_ref[0] = fm
        for c in range(CX):
            mx_ref[c] = mx[:, c * 128:(c + 1) * 128]
        P_ref[0] = zmat(We1_ref, 0)
        Q_ref[0] = zmat(We1_ref, HID)

    return pl.pallas_call(
        body,
        grid=(NEXP, NB),
        in_specs=[
            pl.BlockSpec((CZ, RB, 128), lambda i, r: (0, r, 0)),
            pl.BlockSpec((RB, F_IN), lambda i, r: (r, 0)),
            pl.BlockSpec((1, HID, HID), lambda i, r: (i, 0, 0)),
            pl.BlockSpec((1, 1, HID), lambda i, r: (i, 0, 0)),
            pl.BlockSpec((1, HID, 1), lambda i, r: (i, 0, 0)),
            pl.BlockSpec((1, 1, 1), lambda i, r: (i, 0, 0)),
            pl.BlockSpec((1, 2 * HID, HID), lambda i, r: (i, 0, 0)),
            pl.BlockSpec((1, HID, HID), lambda i, r: (i, 0, 0)),
            pl.BlockSpec((1, 1, HID), lambda i, r: (i, 0, 0)),
            pl.BlockSpec((1, HID, F_IN), lambda i, r: (i, 0, 0)),
            pl.BlockSpec((1, 1, F_IN), lambda i, r: (i, 0, 0)),
            pl.BlockSpec((1, RB, 1), lambda i, r: (i, r, 0)),
            pl.BlockSpec((1, RB, F_IN), lambda i, r: (i, r, 0)),
        ],
        out_specs=[
            pl.BlockSpec((1, RB, 1), lambda i, r: (i, r, 0)),
            pl.BlockSpec((1, RB, F_IN), lambda i, r: (i, r, 0)),
            pl.BlockSpec((CX, RB, 128), lambda i, r: (i, r, 0)),
            pl.BlockSpec((1, RB, HID), lambda i, r: (i, r, 0)),
            pl.BlockSpec((1, RB, HID), lambda i, r: (i, r, 0)),
        ],
        out_shape=[
            jax.ShapeDtypeStruct((NEXP, N, 1), jnp.float32),
            jax.ShapeDtypeStruct((NEXP, N, F_IN), jnp.float32),
            jax.ShapeDtypeStruct((NEXP * CX, N, 128), jnp.float32),
            jax.ShapeDtypeStruct((NEXP, N, HID), jnp.float32),
            jax.ShapeDtypeStruct((NEXP, N, HID), jnp.float32),
        ],
    )(Z_cm.reshape(CZ, N, 128), x, Wn1, bn1.reshape(NEXP, 1, HID), Wn2,
      bn2.reshape(NEXP, 1, 1), We1, Wf1, bf1.reshape(NEXP, 1, HID), Wf2,
      bf2.reshape(NEXP, 1, F_IN), gn, gf)


def _gmp_pool(h_cm, batch3d):
    C = h_cm.shape[0]

    def body(b_ref, h_ref, o_ref, acc_ref, cnt_ref):
        r = pl.program_id(0)
        ids = b_ref[0, 0]
        onehotT = (jax.lax.broadcasted_iota(jnp.int32, (NGRAPH, RB), 0)
                   == ids[None, :]).astype(jnp.float32)

        @pl.when(r == 0)
        def _():
            acc_ref[...] = jnp.zeros_like(acc_ref)
            cnt_ref[...] = jnp.zeros_like(cnt_ref)

        cnt_ref[...] += jnp.sum(onehotT, axis=1, keepdims=True)
        for c in range(C):
            acc_ref[:, c * 128:(c + 1) * 128] += jnp.dot(
                onehotT, h_ref[c], preferred_element_type=jnp.float32)

        @pl.when(r == NB - 1)
        def _():
            o_ref[...] = acc_ref[...] / jnp.maximum(cnt_ref[...], 1.0)

    return pl.pallas_call(
        body,
        grid=(NB,),
        in_specs=[
            pl.BlockSpec((1, 1, RB), lambda r: (r, 0, 0)),
            pl.BlockSpec((C, RB, 128), lambda r: (0, r, 0)),
        ],
        out_specs=pl.BlockSpec((NGRAPH, C * 128), lambda r: (0, 0)),
        out_shape=jax.ShapeDtypeStruct((NGRAPH, C * 128), jnp.float32),
        scratch_shapes=[
            pltpu.VMEM((NGRAPH, C * 128), jnp.float32),
            pltpu.VMEM((NGRAPH, 1), jnp.float32),
        ],
    )(batch3d, h_cm)


def _final_logits(hst_flat, Wc, bc):
    def body(h_ref, W_ref, b_ref, o_ref):
        o_ref[0] = jnp.dot(h_ref[0], W_ref[0],
                           preferred_element_type=jnp.float32) + b_ref[0]

    return pl.pallas_call(
        body,
        grid=(NEXP,),
        in_specs=[
            pl.BlockSpec((1, NGRAPH, HID), lambda i: (i, 0, 0)),
            pl.BlockSpec((1, HID, NCLS), lambda i: (i, 0, 0)),
            pl.BlockSpec((1, 1, NCLS), lambda i: (i, 0, 0)),
        ],
        out_specs=pl.BlockSpec((1, NGRAPH, NCLS), lambda i: (i, 0, 0)),
        out_shape=jax.ShapeDtypeStruct((NEXP, NGRAPH, NCLS), jnp.float32),
    )(hst_flat, Wc, bc.reshape(NEXP, 1, NCLS))


def _gumbel(key, shape):
    u = jax.random.uniform(key, shape, dtype=jnp.float32)
    return -jnp.log(-jnp.log(u + 1e-20) + 1e-20)


def kernel(x, edge_index, batch, ce_W1_0, ce_b1_0, ce_W2_0, ce_b2_0, ce_W1, ce_b1, ce_W2, ce_b2, ce_eps, cl_W1_0, cl_b1_0, cl_W2_0, cl_b2_0, cl_W1, cl_b1, cl_W2, cl_b2, cl_eps, Wn1, bn1, Wn2, bn2, We1, be1, We2, be2, Wf1, bf1, Wf2, bf2, Wc, bc):
    src = edge_index[0]; dst = edge_index[1]
    order = jnp.argsort(dst, stable=True)
    src_s = src[order]; dst_s = dst[order]
    npad = EPAD - E
    src_pad = jnp.concatenate([src_s, jnp.zeros((npad,), jnp.int32)])
    dummy = DUMMY0 + (jnp.arange(E, dtype=jnp.int32) % 16)
    dst_pad_plain = jnp.concatenate(
        [dst_s, jnp.full((npad,), DUMMY0, jnp.int32)])

    x_cm = x.reshape(N, 2, 128).transpose(1, 0, 2)
    h = x_cm
    ce_Ws = [(ce_W1_0, ce_b1_0, ce_W2_0, ce_b2_0), (ce_W1[0], ce_b1[0], ce_W2[0], ce_b2[0]), (ce_W1[1], ce_b1[1], ce_W2[1], ce_b2[1])]
    for l in range(NL):
        c = h.shape[0]
        aggf = _seg_pass(h.reshape(c * N, 128), src_pad, dst_pad_plain, c, 1)
        W1l, b1l, W2l, b2l = ce_Ws[l]
        h = _gin_mlp(h, aggf.reshape(c, N, 128), ce_eps[l], W1l, b1l, W2l, b2l,
                     relu_out=(l < NL - 1), nexp=1)
    Z_cm = h

    mkey = jax.random.key(42)
    g_n = jnp.stack([_gumbel(jax.random.fold_in(mkey, 3 * i), (N, 1)) for i in range(NEXP)])
    g_e = [_gumbel(jax.random.fold_in(mkey, 3 * i + 1), (E, 1)) for i in range(NEXP)]
    g_f = jnp.stack([_gumbel(jax.random.fold_in(mkey, 3 * i + 2), (N, F_IN)) for i in range(NEXP)])

    nm, fm, mx_cm, P, Q = _expert_heads(Z_cm, x, Wn1, bn1, Wn2, bn2, We1,
                                        Wf1, bf1, Wf2, bf2, g_n, g_f)

    em_list = []; dst_lists = []
    for i in range(NEXP):
        ehid = jnp.maximum(P[i][src] + Q[i][dst] + be1[i], 0.0)
        elog = ehid @ We2[i] + be2[i]
        emask = (elog + g_e[i] > 0.0).astype(jnp.float32)
        em_list.append(emask)
        emask_s = emask.reshape(-1)[order]
        dst_adj = jnp.where(emask_s > 0.0, dst_s, dummy)
        dst_lists.append(jnp.concatenate([dst_adj, jnp.full((npad,), DUMMY0, jnp.int32)]))
    dl_all = jnp.concatenate(dst_lists)

    cl_Ws = [(cl_W1_0, cl_b1_0, cl_W2_0, cl_b2_0), (cl_W1[0], cl_b1[0], cl_W2[0], cl_b2[0]), (cl_W1[1], cl_b1[1], cl_W2[1], cl_b2[1])]
    h = mx_cm
    for l in range(NL):
        c = h.shape[0]
        aggf = _seg_pass(h.reshape(c * N, 128), src_pad, dl_all, c, NEXP)
        W1l, b1l, W2l, b2l = cl_Ws[l]
        h = _gin_mlp(h, aggf.reshape(c, N, 128), cl_eps[l], W1l, b1l, W2l, b2l,
                     relu_out=(l < NL - 1), nexp=NEXP)
    mZ_cm = h

    batch3d = batch.reshape(NB, 1, RB)
    hst = _gmp_pool(mZ_cm, batch3d).reshape(NGRAPH, NEXP, HID)
    h_orig = _gmp_pool(Z_cm, batch3d)
    logits = _final_logits(hst.transpose(1, 0, 2), Wc, bc)

    expert_logits = logits.transpose(1, 0, 2)
    h_stable = hst
    node_masks = nm.transpose(1, 0, 2)
    edge_masks = jnp.stack(em_list, axis=1)
    feat_masks = fm.transpose(1, 0, 2)
    return expert_logits, h_stable, h_orig, node_masks, edge_masks, feat_masks


# speedup vs baseline: 1.5188x; 1.0035x over previous
import functools

import jax
import jax.numpy as jnp
from jax import lax
from jax.experimental import pallas as pl
from jax.experimental.pallas import tpu as pltpu
from jax.experimental.pallas import tpu_sc as plsc

N = 10000; E = 160000; F_IN = 256; HID = 512; NL = 3; NEXP = 4; NCLS = 10; NGRAPH = 64

NPAD = 10240
DUMMY0 = N
CH = 128
EPT = 10240
EPAD = 16 * EPT


def _seg_pass(h_flat, src_pad, dst_pad, C, NLIST):
    C2 = C // 2
    CPL = C // NLIST
    NIT = EPT // CH
    NH = NIT // 2

    mesh = plsc.VectorSubcoreMesh(core_axis_name="c", subcore_axis_name="s")

    @functools.partial(
        pl.kernel, mesh=mesh,
        out_type=jax.ShapeDtypeStruct((C * N, 128), jnp.float32),
        scratch_types=[
            pltpu.VMEM((NH, CH), jnp.int32),
            pltpu.VMEM((NH, CH), jnp.int32),
            pltpu.VMEM((CH, 128), jnp.float32),
            pltpu.VMEM((CH, 128), jnp.float32),
            pltpu.VMEM((32, 128), jnp.float32),
            pltpu.VMEM_SHARED((NPAD, 128), jnp.float32),
            pltpu.SemaphoreType.DMA,
            pltpu.SemaphoreType.DMA,
            pltpu.SemaphoreType.DMA,
        ],
    )
    def k(h_hbm, src_hbm, dst_hbm, out_hbm, sidx, didx, rows0, rows1,
          zbuf, acc, sem0, sem1, zsem):
        c = lax.axis_index("c")
        s = lax.axis_index("s")
        zero16 = jnp.zeros((16,), jnp.float32)

        def zfill(j, _):
            zbuf[j // 8, pl.ds((j % 8) * 16, 16)] = zero16
            return 0
        lax.fori_loop(0, 256, zfill, 0)
        rows = (rows0, rows1)
        sems = (sem0, sem1)

        for kk in range(C2):
            chunk = c * C2 + kk
            list_id = chunk // CPL

            zh = [pltpu.async_copy(zbuf, acc.at[pl.ds(s * 640 + j * 32, 32)], zsem)
                  for j in range(20)]
            for h in zh:
                h.wait()
            plsc.subcore_barrier()

            for hh in range(2):
                pltpu.sync_copy(src_hbm.at[pl.ds(s * NIT + hh * NH, NH)], sidx)
                pltpu.sync_copy(
                    dst_hbm.at[pl.ds(list_id * (EPAD // CH) + s * NIT + hh * NH, NH)],
                    didx)

                def adj(j, _):
                    r = j // (CH // 16)
                    col = (j % (CH // 16)) * 16
                    sidx[r, pl.ds(col, 16)] = sidx[r, pl.ds(col, 16)] + chunk * N
                    return 0
                lax.fori_loop(0, NH * (CH // 16), adj, 0)

                for g in range(NH):
                    sl = g % 2
                    pltpu.async_copy(h_hbm.at[sidx.at[g]], rows[sl], sems[sl]).wait()
                    pltpu.sync_copy(rows[sl], acc.at[didx.at[g]], add=True)
            plsc.subcore_barrier()

            offr = jnp.minimum(s * 632, N - 632)
            pltpu.sync_copy(acc.at[pl.ds(offr, 632)],
                            out_hbm.at[pl.ds(chunk * N + offr, 632)])
            plsc.subcore_barrier()

    return k(h_flat, src_pad.reshape(EPAD // CH, CH), dst_pad.reshape(NLIST * (EPAD // CH), CH))


def _to_chunk_major(h):
    n, f = h.shape
    c = f // 128
    return h.reshape(n, c, 128).transpose(1, 0, 2).reshape(c * n, 128)


def _from_chunk_major(hf, c):
    return hf.reshape(c, N, 128).transpose(1, 0, 2).reshape(N, c * 128)


RB = 400
NB = N // RB


def _gin_mlp(h_cm, agg_cm, eps, W1, b1, W2, b2, relu_out, nexp):
    cin = W1.shape[0] // 128
    cout = W2.shape[1] // 128
    eps = eps.reshape(1, 1)

    def body(eps_ref, h_ref, agg_ref, W1_ref, b1_ref, W2_ref, b2_ref, o_ref):
        scale = 1.0 + eps_ref[0, 0]
        h2 = jnp.concatenate(
            [scale * h_ref[c] + agg_ref[c] for c in range(cin)], axis=1)
        hid = jnp.maximum(
            jnp.dot(h2, W1_ref[...], preferred_element_type=jnp.float32)
            + b1_ref[0], 0.0)
        out = jnp.dot(hid, W2_ref[...], preferred_element_type=jnp.float32) + b2_ref[0]
        if relu_out:
            out = jnp.maximum(out, 0.0)
        for c in range(cout):
            o_ref[c] = out[:, c * 128:(c + 1) * 128]

    return pl.pallas_call(
        body,
        grid=(nexp, NB),
        in_specs=[
            pl.BlockSpec(memory_space=pltpu.SMEM),
            pl.BlockSpec((cin, RB, 128), lambda i, r: (i, r, 0)),
            pl.BlockSpec((cin, RB, 128), lambda i, r: (i, r, 0)),
            pl.BlockSpec((cin * 128, W1.shape[1]), lambda i, r: (0, 0)),
            pl.BlockSpec((1, W1.shape[1]), lambda i, r: (0, 0)),
            pl.BlockSpec((W2.shape[0], cout * 128), lambda i, r: (0, 0)),
            pl.BlockSpec((1, cout * 128), lambda i, r: (0, 0)),
        ],
        out_specs=pl.BlockSpec((cout, RB, 128), lambda i, r: (i, r, 0)),
        out_shape=jax.ShapeDtypeStruct((nexp * cout, N, 128), jnp.float32),
    )(eps, h_cm, agg_cm, W1, b1.reshape(1, -1), W2, b2.reshape(1, -1))


def _expert_heads(Z_cm, x, Wn1, bn1, Wn2, bn2, We1, Wf1, bf1, Wf2, bf2,
                  gn, gf):
    CZ = HID // 128
    CX = F_IN // 128

    def body(Z_ref, x_ref, Wn1_ref, bn1_ref, Wn2_ref, bn2_ref, We1_ref,
             Wf1_ref, bf1_ref, Wf2_ref, bf2_ref, gn_ref, gf_ref,
             nm_ref, fm_ref, mx_ref, P_ref, Q_ref):
        Zb = jnp.concatenate([Z_ref[c] for c in range(CZ)], axis=1)

        def zmat(W_ref, r0):
            return jnp.dot(Zb, W_ref[0, pl.ds(r0, HID), :],
                           preferred_element_type=jnp.float32)

        hidN = jnp.maximum(zmat(Wn1_ref, 0) + bn1_ref[0], 0.0)
        nlog = jnp.dot(hidN, Wn2_ref[0], preferred_element_type=jnp.float32) + bn2_ref[0]
        nm = (nlog + gn_ref[0] > 0.0).astype(jnp.float32)
        hidF = jnp.maximum(zmat(Wf1_ref, 0) + bf1_ref[0], 0.0)
        flog = jnp.dot(hidF, Wf2_ref[0], preferred_element_type=jnp.float32) + bf2_ref[0]
        fm = (flog + gf_ref[0] > 0.0).astype(jnp.float32)
        mx = x_ref[...] * nm * fm
        nm_ref[0] = nm
        fm_ref[0] = fm
        for c in range(CX):
            mx_ref[c] = mx[:, c * 128:(c + 1) * 128]
        P_ref[0] = zmat(We1_ref, 0)
        Q_ref[0] = zmat(We1_ref, HID)

    return pl.pallas_call(
        body,
        grid=(NEXP, NB),
        in_specs=[
            pl.BlockSpec((CZ, RB, 128), lambda i, r: (0, r, 0)),
            pl.BlockSpec((RB, F_IN), lambda i, r: (r, 0)),
            pl.BlockSpec((1, HID, HID), lambda i, r: (i, 0, 0)),
            pl.BlockSpec((1, 1, HID), lambda i, r: (i, 0, 0)),
            pl.BlockSpec((1, HID, 1), lambda i, r: (i, 0, 0)),
            pl.BlockSpec((1, 1, 1), lambda i, r: (i, 0, 0)),
            pl.BlockSpec((1, 2 * HID, HID), lambda i, r: (i, 0, 0)),
            pl.BlockSpec((1, HID, HID), lambda i, r: (i, 0, 0)),
            pl.BlockSpec((1, 1, HID), lambda i, r: (i, 0, 0)),
            pl.BlockSpec((1, HID, F_IN), lambda i, r: (i, 0, 0)),
            pl.BlockSpec((1, 1, F_IN), lambda i, r: (i, 0, 0)),
            pl.BlockSpec((1, RB, 1), lambda i, r: (i, r, 0)),
            pl.BlockSpec((1, RB, F_IN), lambda i, r: (i, r, 0)),
        ],
        out_specs=[
            pl.BlockSpec((1, RB, 1), lambda i, r: (i, r, 0)),
            pl.BlockSpec((1, RB, F_IN), lambda i, r: (i, r, 0)),
            pl.BlockSpec((CX, RB, 128), lambda i, r: (i, r, 0)),
            pl.BlockSpec((1, RB, HID), lambda i, r: (i, r, 0)),
            pl.BlockSpec((1, RB, HID), lambda i, r: (i, r, 0)),
        ],
        out_shape=[
            jax.ShapeDtypeStruct((NEXP, N, 1), jnp.float32),
            jax.ShapeDtypeStruct((NEXP, N, F_IN), jnp.float32),
            jax.ShapeDtypeStruct((NEXP * CX, N, 128), jnp.float32),
            jax.ShapeDtypeStruct((NEXP, N, HID), jnp.float32),
            jax.ShapeDtypeStruct((NEXP, N, HID), jnp.float32),
        ],
    )(Z_cm.reshape(CZ, N, 128), x, Wn1, bn1.reshape(NEXP, 1, HID), Wn2,
      bn2.reshape(NEXP, 1, 1), We1, Wf1, bf1.reshape(NEXP, 1, HID), Wf2,
      bf2.reshape(NEXP, 1, F_IN), gn, gf)


def _gmp_pool(h_cm, batch3d):
    C = h_cm.shape[0]

    def body(b_ref, h_ref, o_ref, acc_ref, cnt_ref):
        r = pl.program_id(0)
        ids = b_ref[0, 0]
        onehotT = (jax.lax.broadcasted_iota(jnp.int32, (NGRAPH, RB), 0)
                   == ids[None, :]).astype(jnp.float32)

        @pl.when(r == 0)
        def _():
            acc_ref[...] = jnp.zeros_like(acc_ref)
            cnt_ref[...] = jnp.zeros_like(cnt_ref)

        cnt_ref[...] += jnp.sum(onehotT, axis=1, keepdims=True)
        for c in range(C):
            acc_ref[:, c * 128:(c + 1) * 128] += jnp.dot(
                onehotT, h_ref[c], preferred_element_type=jnp.float32)

        @pl.when(r == NB - 1)
        def _():
            o_ref[...] = acc_ref[...] / jnp.maximum(cnt_ref[...], 1.0)

    return pl.pallas_call(
        body,
        grid=(NB,),
        in_specs=[
            pl.BlockSpec((1, 1, RB), lambda r: (r, 0, 0)),
            pl.BlockSpec((C, RB, 128), lambda r: (0, r, 0)),
        ],
        out_specs=pl.BlockSpec((NGRAPH, C * 128), lambda r: (0, 0)),
        out_shape=jax.ShapeDtypeStruct((NGRAPH, C * 128), jnp.float32),
        scratch_shapes=[
            pltpu.VMEM((NGRAPH, C * 128), jnp.float32),
            pltpu.VMEM((NGRAPH, 1), jnp.float32),
        ],
    )(batch3d, h_cm)


def _final_logits(hst_flat, Wc, bc):
    def body(h_ref, W_ref, b_ref, o_ref):
        o_ref[0] = jnp.dot(h_ref[0], W_ref[0],
                           preferred_element_type=jnp.float32) + b_ref[0]

    return pl.pallas_call(
        body,
        grid=(NEXP,),
        in_specs=[
            pl.BlockSpec((1, NGRAPH, HID), lambda i: (i, 0, 0)),
            pl.BlockSpec((1, HID, NCLS), lambda i: (i, 0, 0)),
            pl.BlockSpec((1, 1, NCLS), lambda i: (i, 0, 0)),
        ],
        out_specs=pl.BlockSpec((1, NGRAPH, NCLS), lambda i: (i, 0, 0)),
        out_shape=jax.ShapeDtypeStruct((NEXP, NGRAPH, NCLS), jnp.float32),
    )(hst_flat, Wc, bc.reshape(NEXP, 1, NCLS))


def _gumbel(key, shape):
    u = jax.random.uniform(key, shape, dtype=jnp.float32)
    return -jnp.log(-jnp.log(u + 1e-20) + 1e-20)


def kernel(x, edge_index, batch, ce_W1_0, ce_b1_0, ce_W2_0, ce_b2_0, ce_W1, ce_b1, ce_W2, ce_b2, ce_eps, cl_W1_0, cl_b1_0, cl_W2_0, cl_b2_0, cl_W1, cl_b1, cl_W2, cl_b2, cl_eps, Wn1, bn1, Wn2, bn2, We1, be1, We2, be2, Wf1, bf1, Wf2, bf2, Wc, bc):
    src = edge_index[0]; dst = edge_index[1]
    order = jnp.argsort(dst, stable=True)
    src_s = src[order]; dst_s = dst[order]
    npad = EPAD - E
    src_pad = jnp.concatenate([src_s, jnp.zeros((npad,), jnp.int32)])
    dummy = DUMMY0 + (jnp.arange(E, dtype=jnp.int32) % 16)
    dst_pad_plain = jnp.concatenate(
        [dst_s, jnp.full((npad,), DUMMY0, jnp.int32)])

    x_cm = x.reshape(N, 2, 128).transpose(1, 0, 2)
    h = x_cm
    ce_Ws = [(ce_W1_0, ce_b1_0, ce_W2_0, ce_b2_0), (ce_W1[0], ce_b1[0], ce_W2[0], ce_b2[0]), (ce_W1[1], ce_b1[1], ce_W2[1], ce_b2[1])]
    for l in range(NL):
        c = h.shape[0]
        aggf = _seg_pass(h.reshape(c * N, 128), src_pad, dst_pad_plain, c, 1)
        W1l, b1l, W2l, b2l = ce_Ws[l]
        h = _gin_mlp(h, aggf.reshape(c, N, 128), ce_eps[l], W1l, b1l, W2l, b2l,
                     relu_out=(l < NL - 1), nexp=1)
    Z_cm = h

    mkey = jax.random.key(42)
    g_n = jnp.stack([_gumbel(jax.random.fold_in(mkey, 3 * i), (N, 1)) for i in range(NEXP)])
    g_e = [_gumbel(jax.random.fold_in(mkey, 3 * i + 1), (E, 1)) for i in range(NEXP)]
    g_f = jnp.stack([_gumbel(jax.random.fold_in(mkey, 3 * i + 2), (N, F_IN)) for i in range(NEXP)])

    nm, fm, mx_cm, P, Q = _expert_heads(Z_cm, x, Wn1, bn1, Wn2, bn2, We1,
                                        Wf1, bf1, Wf2, bf2, g_n, g_f)

    em_list = []; dst_lists = []
    for i in range(NEXP):
        ehid = jnp.maximum(P[i][src] + Q[i][dst] + be1[i], 0.0)
        elog = ehid @ We2[i] + be2[i]
        emask = (elog + g_e[i] > 0.0).astype(jnp.float32)
        em_list.append(emask)
        emask_s = emask.reshape(-1)[order]
        dst_adj = jnp.where(emask_s > 0.0, dst_s, dummy)
        dst_lists.append(jnp.concatenate([dst_adj, jnp.full((npad,), DUMMY0, jnp.int32)]))
    dl_all = jnp.concatenate(dst_lists)

    cl_Ws = [(cl_W1_0, cl_b1_0, cl_W2_0, cl_b2_0), (cl_W1[0], cl_b1[0], cl_W2[0], cl_b2[0]), (cl_W1[1], cl_b1[1], cl_W2[1], cl_b2[1])]
    h = mx_cm
    for l in range(NL):
        c = h.shape[0]
        aggf = _seg_pass(h.reshape(c * N, 128), src_pad, dl_all, c, NEXP)
        W1l, b1l, W2l, b2l = cl_Ws[l]
        h = _gin_mlp(h, aggf.reshape(c, N, 128), cl_eps[l], W1l, b1l, W2l, b2l,
                     relu_out=(l < NL - 1), nexp=NEXP)
    mZ_cm = h

    batch3d = batch.reshape(NB, 1, RB)
    hst = _gmp_pool(mZ_cm, batch3d).reshape(NGRAPH, NEXP, HID)
    h_orig = _gmp_pool(Z_cm, batch3d)
    logits = _final_logits(hst.transpose(1, 0, 2), Wc, bc)

    expert_logits = logits.transpose(1, 0, 2)
    h_stable = hst
    node_masks = nm.transpose(1, 0, 2)
    edge_masks = jnp.stack(em_list, axis=1)
    feat_masks = fm.transpose(1, 0, 2)
    return expert_logits, h_stable, h_orig, node_masks, edge_masks, feat_masks
